# baseline (speedup 1.0000x reference)
import jax
import jax.numpy as jnp
from jax import lax
from jax.experimental import pallas as pl
from jax.experimental.pallas import tpu as pltpu
from jax.experimental.pallas import tpu_sc as plsc

_BATCH = 16384
_D = 32
_NW = 32
_BPW = _BATCH // _NW
_WIN = 512
_CPW = 244 * 128
_TAIL0 = 7812 * 128
_LCAP = 1024
_CCAP = 64
_STRIP = _BATCH + 8
_DUMP = _BATCH

def _splat(v):
    return jnp.full((16,), v, jnp.int32)


def _prefix16(m, sbuf):
    iota = lax.iota(jnp.int32, 16)
    x = m
    for d in (1, 2, 4, 8):
        sbuf[pl.ds(0, 16)] = x
        shifted = plsc.load_gather(sbuf, [jnp.maximum(iota - d, 0)])
        x = x + jnp.where(iota >= d, shifted, 0)
    return x


def _scan_body(utab_hbm, itab_hbm, uidx_hbm, iidx_hbm,
               ustrip_hbm, istrip_hbm,
               idxu_v, idxi_v, jloc_u, rloc_u, jloc_i, rloc_i,
               ubuf_v, ibuf_v, urec_v, irec_v,
               jc_u, rc_u, jc_i, rc_i, sbuf_v, sem_u, sem_i):
    wid = lax.axis_index("s") * 2 + lax.axis_index("c")
    lo = wid * _CPW
    hi = jnp.where(wid == _NW - 1, _TAIL0, lo + _CPW)
    nwin = (hi - lo) // _WIN
    iota16 = lax.iota(jnp.int32, 16)

    for k in range(_LCAP // 16):
        ko = pl.ds(k * 16, 16)
        jloc_u[ko] = _splat(-1)
        rloc_u[ko] = _splat(-1)
        jloc_i[ko] = _splat(-1)
        rloc_i[ko] = _splat(-1)

    def scan_block(b, cnts):
        pltpu.sync_copy(uidx_hbm.at[pl.ds(b * 2048, 2048)], idxu_v)
        pltpu.sync_copy(iidx_hbm.at[pl.ds(b * 2048, 2048)], idxi_v)

        def grp(g, cnts):
            cu, ci = cnts
            jbase = b * 2048 + g * 16
            o = pl.ds(g * 16, 16)
            ru = idxu_v[o]
            mu = (ru >= lo) & (ru < hi)
            su = (cu + _prefix16(jnp.where(mu, 1, 0), sbuf_v) - 1) & (_LCAP - 1)
            plsc.store_scatter(rloc_u, [su], ru, mask=mu)
            plsc.store_scatter(jloc_u, [su], jbase + iota16, mask=mu)
            cu = cu + plsc.all_reduce_population_count(mu)
            ri = idxi_v[o]
            mi = (ri >= lo) & (ri < hi)
            si = (ci + _prefix16(jnp.where(mi, 1, 0), sbuf_v) - 1) & (_LCAP - 1)
            plsc.store_scatter(rloc_i, [si], ri, mask=mi)
            plsc.store_scatter(jloc_i, [si], jbase + iota16, mask=mi)
            ci = ci + plsc.all_reduce_population_count(mi)
            return (cu, ci)

        return lax.fori_loop(0, 128, grp, cnts)

    cu, ci = lax.fori_loop(0, _BATCH // 2048, scan_block,
                           (_splat(0), _splat(0)))
    ngu = (jnp.max(cu) + 15) >> 4
    ngi = (jnp.max(ci) + 15) >> 4

    def window(c, _):
        rbase = pl.multiple_of(lo + c * _WIN, 128)
        pltpu.sync_copy(utab_hbm.at[:, pl.ds(rbase, _WIN)], ubuf_v)
        pltpu.sync_copy(itab_hbm.at[:, pl.ds(rbase, _WIN)], ibuf_v)

        def one_table(ng, jloc, rloc, buf, rec, jc, rc):
            for k in range(_CCAP // 16):
                jc[pl.ds(k * 16, 16)] = _splat(_DUMP)
                rc[pl.ds(k * 16, 16)] = _splat(0)

            def minis(s, cc):
                so = pl.ds(pl.multiple_of(s * 16, 8), 16)
                r = rloc[so]
                j = jloc[so]
                m = (r >= rbase) & (r < rbase + _WIN)
                sl = (cc + _prefix16(jnp.where(m, 1, 0), sbuf_v) - 1) & (_CCAP - 1)
                plsc.store_scatter(rc, [sl], r - rbase, mask=m)
                plsc.store_scatter(jc, [sl], j, mask=m)
                return cc + plsc.all_reduce_population_count(m)

            lax.fori_loop(0, ng, minis, _splat(0))

            for t in range(_CCAP // 16):
                to = pl.ds(t * 16, 16)
                cols = rc[to]
                rows = t * 16 + iota16
                for f in range(_D):
                    vals = plsc.load_gather(buf, [_splat(f), cols])
                    plsc.store_scatter(rec, [rows, _splat(f)], vals)

        one_table(ngu, jloc_u, rloc_u, ubuf_v, urec_v, jc_u, rc_u)
        one_table(ngi, jloc_i, rloc_i, ibuf_v, irec_v, jc_i, rc_i)
        wu = pltpu.async_copy(urec_v, ustrip_hbm.at[jc_u], sem_u)
        wi = pltpu.async_copy(irec_v, istrip_hbm.at[jc_i], sem_i)
        wu.wait()
        wi.wait()
        return _

    lax.fori_loop(0, nwin, window, None)


def _merge_body(ustrip_hbm, istrip_hbm, uidx_hbm, iidx_hbm,
                utail_hbm, itail_hbm, out_hbm,
                sh_idx, idxu_s, idxi_s,
                uchunk_v, ichunk_v, utail_v, itail_v, prod_v):
    wid = lax.axis_index("s") * 2 + lax.axis_index("c")
    base = pl.multiple_of(wid * _BPW, 128)

    @pl.when(lax.axis_index("s") == 0)
    def _fill():
        pltpu.sync_copy(uidx_hbm, sh_idx.at[0])
        pltpu.sync_copy(iidx_hbm, sh_idx.at[1])

    plsc.subcore_barrier()
    pltpu.sync_copy(sh_idx.at[0, pl.ds(base, _BPW)], idxu_s)
    pltpu.sync_copy(sh_idx.at[1, pl.ds(base, _BPW)], idxi_s)
    pltpu.sync_copy(utail_hbm, utail_v)
    pltpu.sync_copy(itail_hbm, itail_v)

    for q in range(_BPW // 128):
        qbase = pl.multiple_of(base + q * 128, 8)
        pltpu.sync_copy(ustrip_hbm.at[pl.ds(qbase, 128), :], uchunk_v)
        pltpu.sync_copy(istrip_hbm.at[pl.ds(qbase, 128), :], ichunk_v)

        def slot(s2, _):
            ju = idxu_s[q * 128 + s2]
            ji = idxi_s[q * 128 + s2]
            mu = jnp.full((16,), jnp.where(ju >= _TAIL0, -1, 0), jnp.int32)
            mi = jnp.full((16,), jnp.where(ji >= _TAIL0, -1, 0), jnp.int32)
            tu = (ju - _TAIL0) & 63
            ti = (ji - _TAIL0) & 63
            o0 = pl.ds(0, 16)
            o1 = pl.ds(16, 16)

            def blend(tail, chunk, m):
                tb = plsc.bitcast(tail, jnp.int32)
                cb = plsc.bitcast(chunk, jnp.int32)
                return plsc.bitcast((tb & m) | (cb & ~m), jnp.float32)

            u0 = blend(utail_v[tu, o0], uchunk_v[s2, o0], mu)
            u1 = blend(utail_v[tu, o1], uchunk_v[s2, o1], mu)
            v0 = blend(itail_v[ti, o0], ichunk_v[s2, o0], mi)
            v1 = blend(itail_v[ti, o1], ichunk_v[s2, o1], mi)
            prod_v[s2, o0] = u0 * v0
            prod_v[s2, o1] = u1 * v1
            return _

        lax.fori_loop(0, 128, slot, None)
        pltpu.sync_copy(prod_v, out_hbm.at[pl.ds(qbase, 128), :])


@jax.jit
def kernel(user_indices, item_indices, user_table, item_table):
    mesh = plsc.VectorSubcoreMesh(core_axis_name="c", subcore_axis_name="s")
    uidx = user_indices.astype(jnp.int32)
    iidx = item_indices.astype(jnp.int32)
    utab_t = user_table.T
    itab_t = item_table.T
    utail = user_table[_TAIL0:, :]
    itail = item_table[_TAIL0:, :]

    k1 = pl.kernel(
        _scan_body,
        out_type=(
            jax.ShapeDtypeStruct((_STRIP, 128), jnp.float32),
            jax.ShapeDtypeStruct((_STRIP, 128), jnp.float32),
        ),
        mesh=mesh,
        scratch_types=[
            pltpu.VMEM((2048,), jnp.int32),
            pltpu.VMEM((2048,), jnp.int32),
            pltpu.VMEM((_LCAP,), jnp.int32),
            pltpu.VMEM((_LCAP,), jnp.int32),
            pltpu.VMEM((_LCAP,), jnp.int32),
            pltpu.VMEM((_LCAP,), jnp.int32),
            pltpu.VMEM((_D, _WIN), jnp.float32),
            pltpu.VMEM((_D, _WIN), jnp.float32),
            pltpu.VMEM((_CCAP, 128), jnp.float32),
            pltpu.VMEM((_CCAP, 128), jnp.float32),
            pltpu.VMEM((_CCAP,), jnp.int32),
            pltpu.VMEM((_CCAP,), jnp.int32),
            pltpu.VMEM((_CCAP,), jnp.int32),
            pltpu.VMEM((_CCAP,), jnp.int32),
            pltpu.VMEM((32,), jnp.int32),
            pltpu.SemaphoreType.DMA,
            pltpu.SemaphoreType.DMA,
        ],
        compiler_params=pltpu.CompilerParams(
            disable_bounds_checks=True, needs_layout_passes=False),
    )
    ustrip, istrip = k1(utab_t, itab_t, uidx, iidx)

    k2 = pl.kernel(
        _merge_body,
        out_type=jax.ShapeDtypeStruct((_BATCH, 128), jnp.float32),
        mesh=mesh,
        scratch_types=[
            pltpu.VMEM_SHARED((2, _BATCH), jnp.int32),
            pltpu.SMEM((_BPW,), jnp.int32),
            pltpu.SMEM((_BPW,), jnp.int32),
            pltpu.VMEM((128, 128), jnp.float32),
            pltpu.VMEM((128, 128), jnp.float32),
            pltpu.VMEM((64, _D), jnp.float32),
            pltpu.VMEM((64, _D), jnp.float32),
            pltpu.VMEM((128, 128), jnp.float32),
        ],
        compiler_params=pltpu.CompilerParams(
            disable_bounds_checks=True, needs_layout_passes=False),
    )
    padded = k2(ustrip, istrip, uidx, iidx, utail, itail)
    return padded[:, :_D]
